# baseline (speedup 1.0000x reference)
import functools

import jax
import jax.numpy as jnp
from jax import lax
from jax.experimental import pallas as pl
from jax.experimental.pallas import tpu as pltpu
from jax.experimental.pallas import tpu_sc as plsc

_N = 100000
_H = 128
_G = 512
_BLK = 12544
_NPAD = 100352

_NS = 16
_CHUNK = 6400
_LAST = _N - (_NS - 1) * _CHUNK
_IT_FULL = _CHUNK // 16
_IT_LAST = _LAST // 16
_STRIDE = 513


def _scores_body(off_blocks, h_ref, p_ref, w_ref, o_ref):
    dn = (((1,), (1,)), ((), ()))
    s = (lax.dot_general(w_ref[:, :_H], h_ref[...], dn,
                         preferred_element_type=jnp.float32)
         + lax.dot_general(w_ref[:, _H:], p_ref[...], dn,
                           preferred_element_type=jnp.float32))
    i = pl.program_id(0) + off_blocks
    col = i * _BLK + lax.broadcasted_iota(jnp.int32, (1, _BLK), 1)
    o_ref[...] = jnp.where(col < _N, s, 0.0)


def _node_scores(h, p, W, off_blocks, nblocks):
    return pl.pallas_call(
        functools.partial(_scores_body, off_blocks),
        grid=(nblocks,),
        in_specs=[
            pl.BlockSpec((_BLK, _H), lambda i: (i + off_blocks, 0)),
            pl.BlockSpec((_BLK, _H), lambda i: (i + off_blocks, 0)),
            pl.BlockSpec((1, 2 * _H), lambda i: (0, 0)),
        ],
        out_specs=pl.BlockSpec((1, _BLK), lambda i: (0, i)),
        out_shape=jax.ShapeDtypeStruct((1, nblocks * _BLK), jnp.float32),
    )(h, p, W)


def _make_seg_body(sbase0, ibase0, chunk, last, vector_prev):
    it_full = chunk // 16
    it_last = last // 16

    def body(s_hbm, ids_hbm, prev_hbm, out_hbm, sv, iv, acc, accg, pv, shared):
        sid = lax.axis_index("s")
        sbase = sbase0 + sid * chunk
        ibase = ibase0 + sid * chunk
        is_last = sid == _NS - 1

        if last == chunk:
            pltpu.sync_copy(s_hbm.at[pl.ds(sbase, chunk)], sv)
            pltpu.sync_copy(ids_hbm.at[pl.ds(ibase, chunk)], iv)
        else:
            @pl.when(jnp.logical_not(is_last))
            def _():
                pltpu.sync_copy(s_hbm.at[pl.ds(sbase, chunk)], sv)
                pltpu.sync_copy(ids_hbm.at[pl.ds(ibase, chunk)], iv)

            @pl.when(is_last)
            def _():
                pltpu.sync_copy(s_hbm.at[pl.ds(sbase, last)],
                                sv.at[pl.ds(0, last)])
                pltpu.sync_copy(ids_hbm.at[pl.ds(ibase, last)],
                                iv.at[pl.ds(0, last)])

        zeros16 = jnp.zeros((16,), jnp.float32)

        def _zero(i, c):
            acc[pl.ds(i * 16, 16)] = zeros16
            return c

        lax.fori_loop(0, _STRIDE, _zero, 0, unroll=16)

        lane_off = lax.broadcasted_iota(jnp.int32, (16,), 0) * _STRIDE

        def _scat(i, c):
            idx = iv[pl.ds(i * 16, 16)] + lane_off
            vals = sv[pl.ds(i * 16, 16)]
            plsc.addupdate_scatter(acc, [idx], vals)
            return c

        if last == chunk:
            lax.fori_loop(0, it_full, _scat, 0, unroll=8)
        else:
            @pl.when(jnp.logical_not(is_last))
            def _():
                lax.fori_loop(0, it_full, _scat, 0, unroll=8)

            @pl.when(is_last)
            def _():
                lax.fori_loop(0, it_last, _scat, 0, unroll=8)

        def _red(j, c):
            v = acc[pl.ds(j * 16, 16)]
            for l in range(1, 16):
                v = v + acc[pl.ds(l * _STRIDE + j * 16, 16)]
            accg[pl.ds(j * 16, 16)] = v
            return c

        lax.fori_loop(0, _G // 16, _red, 0)

        pltpu.sync_copy(accg, shared.at[pl.ds(sid * _G, _G)])
        plsc.subcore_barrier()

        @pl.when(sid == 0)
        def _():
            pltpu.sync_copy(shared, acc.at[pl.ds(0, 16 * _G)])
            if vector_prev:
                pltpu.sync_copy(prev_hbm, pv)
            else:
                pltpu.sync_copy(prev_hbm, pv.at[pl.ds(0, 1)])
            bias = None if vector_prev else pv[pl.ds(0, 16)][0]

            def _red2(j, c):
                if vector_prev:
                    v = pv[pl.ds(j * 16, 16)]
                else:
                    v = jnp.zeros((16,), jnp.float32) + bias
                for l in range(16):
                    v = v + acc[pl.ds(l * _G + j * 16, 16)]
                accg[pl.ds(j * 16, 16)] = v
                return c

            lax.fori_loop(0, _G // 16, _red2, 0)
            pltpu.sync_copy(accg, out_hbm)

    return body


def _make_seg(sbase0, ibase0, chunk, last, vector_prev):
    mesh = plsc.VectorSubcoreMesh(
        core_axis_name="c", subcore_axis_name="s", num_cores=1)
    return pl.kernel(
        _make_seg_body(sbase0, ibase0, chunk, last, vector_prev),
        out_type=jax.ShapeDtypeStruct((_G,), jnp.float32),
        mesh=mesh,
        scratch_types=[
            pltpu.VMEM((chunk,), jnp.float32),
            pltpu.VMEM((chunk,), jnp.int32),
            pltpu.VMEM((16 * _STRIDE,), jnp.float32),
            pltpu.VMEM((_G,), jnp.float32),
            pltpu.VMEM((_G if vector_prev else 16,), jnp.float32),
            pltpu.VMEM_SHARED((16 * _G,), jnp.float32),
        ],
        compiler_params=pltpu.CompilerParams(needs_layout_passes=False),
    )


_HALF = 50176
_CHUNK_B = 3120
_LAST_B = _N - _HALF - (_NS - 1) * _CHUNK_B


def kernel(h, p, h_batch, W, b):
    ids = h_batch.astype(jnp.int32)
    s_a = _node_scores(h, p, W, 0, 4).reshape(_HALF)
    s_b = _node_scores(h, p, W, 4, 4).reshape(_HALF)
    part_a = _make_seg(0, 0, 3136, 3136, False)(s_a, ids, b)
    return _make_seg(0, _HALF, _CHUNK_B, _LAST_B, True)(s_b, ids, part_a)


# speedup vs baseline: 1.0463x; 1.0463x over previous
import functools

import jax
import jax.numpy as jnp
from jax import lax
from jax.experimental import pallas as pl
from jax.experimental.pallas import tpu as pltpu
from jax.experimental.pallas import tpu_sc as plsc

_N = 100000
_H = 128
_G = 512
_BLK = 10240
_NPAD = 102400

_NS = 16
_CHUNK = 6400
_LAST = _N - (_NS - 1) * _CHUNK
_IT_FULL = _CHUNK // 16
_IT_LAST = _LAST // 16
_STRIDE = 513


def _scores_body(off_blocks, h_ref, p_ref, w_ref, o_ref):
    dn = (((1,), (1,)), ((), ()))
    s = (lax.dot_general(w_ref[:, :_H], h_ref[...], dn,
                         preferred_element_type=jnp.float32)
         + lax.dot_general(w_ref[:, _H:], p_ref[...], dn,
                           preferred_element_type=jnp.float32))
    i = pl.program_id(0) + off_blocks
    col = i * _BLK + lax.broadcasted_iota(jnp.int32, (1, _BLK), 1)
    o_ref[...] = jnp.where(col < _N, s, 0.0)


def _node_scores(h, p, W, off_blocks, nblocks):
    return pl.pallas_call(
        functools.partial(_scores_body, off_blocks),
        grid=(nblocks,),
        in_specs=[
            pl.BlockSpec((_BLK, _H), lambda i: (i + off_blocks, 0)),
            pl.BlockSpec((_BLK, _H), lambda i: (i + off_blocks, 0)),
            pl.BlockSpec((1, 2 * _H), lambda i: (0, 0)),
        ],
        out_specs=pl.BlockSpec((1, _BLK), lambda i: (0, i)),
        out_shape=jax.ShapeDtypeStruct((1, nblocks * _BLK), jnp.float32),
    )(h, p, W)


def _make_seg_body(sbase0, ibase0, chunk, last, vector_prev):
    it_full = chunk // 16
    it_last = last // 16

    def body(s_hbm, ids_hbm, prev_hbm, out_hbm, sv, iv, acc, accg, pv, shared):
        sid = lax.axis_index("s")
        sbase = sbase0 + sid * chunk
        ibase = ibase0 + sid * chunk
        is_last = sid == _NS - 1

        if last == chunk:
            pltpu.sync_copy(s_hbm.at[pl.ds(sbase, chunk)], sv)
            pltpu.sync_copy(ids_hbm.at[pl.ds(ibase, chunk)], iv)
        else:
            @pl.when(jnp.logical_not(is_last))
            def _():
                pltpu.sync_copy(s_hbm.at[pl.ds(sbase, chunk)], sv)
                pltpu.sync_copy(ids_hbm.at[pl.ds(ibase, chunk)], iv)

            @pl.when(is_last)
            def _():
                pltpu.sync_copy(s_hbm.at[pl.ds(sbase, last)],
                                sv.at[pl.ds(0, last)])
                pltpu.sync_copy(ids_hbm.at[pl.ds(ibase, last)],
                                iv.at[pl.ds(0, last)])

        zeros16 = jnp.zeros((16,), jnp.float32)

        def _zero(i, c):
            acc[pl.ds(i * 16, 16)] = zeros16
            return c

        lax.fori_loop(0, _STRIDE, _zero, 0, unroll=16)

        lane_off = lax.broadcasted_iota(jnp.int32, (16,), 0) * _STRIDE

        def _scat(i, c):
            idx = iv[pl.ds(i * 16, 16)] + lane_off
            vals = sv[pl.ds(i * 16, 16)]
            plsc.addupdate_scatter(acc, [idx], vals)
            return c

        if last == chunk:
            lax.fori_loop(0, it_full, _scat, 0, unroll=8)
        else:
            @pl.when(jnp.logical_not(is_last))
            def _():
                lax.fori_loop(0, it_full, _scat, 0, unroll=8)

            @pl.when(is_last)
            def _():
                lax.fori_loop(0, it_last, _scat, 0, unroll=8)

        def _red(j, c):
            v = acc[pl.ds(j * 16, 16)]
            for l in range(1, 16):
                v = v + acc[pl.ds(l * _STRIDE + j * 16, 16)]
            accg[pl.ds(j * 16, 16)] = v
            return c

        lax.fori_loop(0, _G // 16, _red, 0)

        pltpu.sync_copy(accg, shared.at[pl.ds(sid * _G, _G)])
        plsc.subcore_barrier()

        @pl.when(sid == 0)
        def _():
            pltpu.sync_copy(shared, acc.at[pl.ds(0, 16 * _G)])
            if vector_prev:
                pltpu.sync_copy(prev_hbm, pv)
            else:
                pltpu.sync_copy(prev_hbm, pv.at[pl.ds(0, 1)])
            bias = None if vector_prev else pv[pl.ds(0, 16)][0]

            def _red2(j, c):
                if vector_prev:
                    v = pv[pl.ds(j * 16, 16)]
                else:
                    v = jnp.zeros((16,), jnp.float32) + bias
                for l in range(16):
                    v = v + acc[pl.ds(l * _G + j * 16, 16)]
                accg[pl.ds(j * 16, 16)] = v
                return c

            lax.fori_loop(0, _G // 16, _red2, 0)
            pltpu.sync_copy(accg, out_hbm)

    return body


def _make_seg(sbase0, ibase0, chunk, last, vector_prev):
    mesh = plsc.VectorSubcoreMesh(
        core_axis_name="c", subcore_axis_name="s", num_cores=1)
    return pl.kernel(
        _make_seg_body(sbase0, ibase0, chunk, last, vector_prev),
        out_type=jax.ShapeDtypeStruct((_G,), jnp.float32),
        mesh=mesh,
        scratch_types=[
            pltpu.VMEM((chunk,), jnp.float32),
            pltpu.VMEM((chunk,), jnp.int32),
            pltpu.VMEM((16 * _STRIDE,), jnp.float32),
            pltpu.VMEM((_G,), jnp.float32),
            pltpu.VMEM((_G if vector_prev else 16,), jnp.float32),
            pltpu.VMEM_SHARED((16 * _G,), jnp.float32),
        ],
        compiler_params=pltpu.CompilerParams(needs_layout_passes=False),
    )


_HALF = 50176
_CHUNK_B = 3120
_LAST_B = _N - _HALF - (_NS - 1) * _CHUNK_B


def kernel(h, p, h_batch, W, b):
    ids = h_batch.astype(jnp.int32)
    s = _node_scores(h, p, W, 0, _NPAD // _BLK).reshape(_NPAD)
    return _make_seg(0, 0, _CHUNK, _LAST, False)(s, ids, b)


# speedup vs baseline: 1.0658x; 1.0187x over previous
import functools

import jax
import jax.numpy as jnp
from jax import lax
from jax.experimental import pallas as pl
from jax.experimental.pallas import tpu as pltpu
from jax.experimental.pallas import tpu_sc as plsc

_N = 100000
_H = 128
_G = 512
_BLK = 10240
_NPAD = 102400

_NS = 16
_CHUNK = 6400
_LAST = _N - (_NS - 1) * _CHUNK
_IT_FULL = _CHUNK // 16
_IT_LAST = _LAST // 16
_STRIDE = 513


def _scores_body(off_blocks, h_ref, p_ref, w_ref, o_ref):
    dn = (((1,), (1,)), ((), ()))
    s = (lax.dot_general(w_ref[:, :_H], h_ref[...], dn,
                         preferred_element_type=jnp.float32)
         + lax.dot_general(w_ref[:, _H:], p_ref[...], dn,
                           preferred_element_type=jnp.float32))
    i = pl.program_id(0) + off_blocks
    col = i * _BLK + lax.broadcasted_iota(jnp.int32, (1, _BLK), 1)
    o_ref[...] = jnp.where(col < _N, s, 0.0)


def _node_scores(h, p, W, off_blocks, nblocks):
    return pl.pallas_call(
        functools.partial(_scores_body, off_blocks),
        grid=(nblocks,),
        in_specs=[
            pl.BlockSpec((_BLK, _H), lambda i: (i + off_blocks, 0)),
            pl.BlockSpec((_BLK, _H), lambda i: (i + off_blocks, 0)),
            pl.BlockSpec((1, 2 * _H), lambda i: (0, 0)),
        ],
        out_specs=pl.BlockSpec((1, _BLK), lambda i: (0, i)),
        out_shape=jax.ShapeDtypeStruct((1, nblocks * _BLK), jnp.float32),
    )(h, p, W)


def _make_seg_body(sbase0, ibase0, chunk, last, vector_prev):
    it_full = chunk // 16
    it_last = last // 16

    def body(s_hbm, ids_hbm, prev_hbm, out_hbm, sv, iv, acc, accg, pv,
             shared, sem_s, sem_i):
        sid = lax.axis_index("s")
        sbase = sbase0 + sid * chunk
        ibase = ibase0 + sid * chunk
        is_last = sid == _NS - 1

        if last == chunk:
            pltpu.async_copy(s_hbm.at[pl.ds(sbase, chunk)], sv, sem_s)
            pltpu.async_copy(ids_hbm.at[pl.ds(ibase, chunk)], iv, sem_i)
        else:
            @pl.when(jnp.logical_not(is_last))
            def _():
                pltpu.async_copy(s_hbm.at[pl.ds(sbase, chunk)], sv, sem_s)
                pltpu.async_copy(ids_hbm.at[pl.ds(ibase, chunk)], iv, sem_i)

            @pl.when(is_last)
            def _():
                pltpu.async_copy(s_hbm.at[pl.ds(sbase, last)],
                                 sv.at[pl.ds(0, last)], sem_s)
                pltpu.async_copy(ids_hbm.at[pl.ds(ibase, last)],
                                 iv.at[pl.ds(0, last)], sem_i)

        zeros16 = jnp.zeros((16,), jnp.float32)

        def _zero(i, c):
            acc[pl.ds(i * 16, 16)] = zeros16
            return c

        lax.fori_loop(0, _STRIDE, _zero, 0, unroll=16)

        if last == chunk:
            pltpu.make_async_copy(s_hbm.at[pl.ds(sbase, chunk)], sv,
                                  sem_s).wait()
            pltpu.make_async_copy(ids_hbm.at[pl.ds(ibase, chunk)], iv,
                                  sem_i).wait()
        else:
            @pl.when(jnp.logical_not(is_last))
            def _():
                pltpu.make_async_copy(s_hbm.at[pl.ds(sbase, chunk)], sv,
                                      sem_s).wait()
                pltpu.make_async_copy(ids_hbm.at[pl.ds(ibase, chunk)], iv,
                                      sem_i).wait()

            @pl.when(is_last)
            def _():
                pltpu.make_async_copy(s_hbm.at[pl.ds(sbase, last)],
                                      sv.at[pl.ds(0, last)], sem_s).wait()
                pltpu.make_async_copy(ids_hbm.at[pl.ds(ibase, last)],
                                      iv.at[pl.ds(0, last)], sem_i).wait()

        lane_off = lax.broadcasted_iota(jnp.int32, (16,), 0) * _STRIDE

        def _scat(i, c):
            idx = iv[pl.ds(i * 16, 16)] + lane_off
            vals = sv[pl.ds(i * 16, 16)]
            plsc.addupdate_scatter(acc, [idx], vals)
            return c

        if last == chunk:
            lax.fori_loop(0, it_full, _scat, 0, unroll=8)
        else:
            @pl.when(jnp.logical_not(is_last))
            def _():
                lax.fori_loop(0, it_full, _scat, 0, unroll=8)

            @pl.when(is_last)
            def _():
                lax.fori_loop(0, it_last, _scat, 0, unroll=8)

        def _red(j, c):
            v = acc[pl.ds(j * 16, 16)]
            for l in range(1, 16):
                v = v + acc[pl.ds(l * _STRIDE + j * 16, 16)]
            accg[pl.ds(j * 16, 16)] = v
            return c

        lax.fori_loop(0, _G // 16, _red, 0)

        pltpu.sync_copy(accg, shared.at[pl.ds(sid * _G, _G)])
        plsc.subcore_barrier()

        @pl.when(sid == 0)
        def _():
            pltpu.sync_copy(shared, acc.at[pl.ds(0, 16 * _G)])
            if vector_prev:
                pltpu.sync_copy(prev_hbm, pv)
            else:
                pltpu.sync_copy(prev_hbm, pv.at[pl.ds(0, 1)])
            bias = None if vector_prev else pv[pl.ds(0, 16)][0]

            def _red2(j, c):
                if vector_prev:
                    v = pv[pl.ds(j * 16, 16)]
                else:
                    v = jnp.zeros((16,), jnp.float32) + bias
                for l in range(16):
                    v = v + acc[pl.ds(l * _G + j * 16, 16)]
                accg[pl.ds(j * 16, 16)] = v
                return c

            lax.fori_loop(0, _G // 16, _red2, 0)
            pltpu.sync_copy(accg, out_hbm)

    return body


def _make_seg(sbase0, ibase0, chunk, last, vector_prev):
    mesh = plsc.VectorSubcoreMesh(
        core_axis_name="c", subcore_axis_name="s", num_cores=1)
    return pl.kernel(
        _make_seg_body(sbase0, ibase0, chunk, last, vector_prev),
        out_type=jax.ShapeDtypeStruct((_G,), jnp.float32),
        mesh=mesh,
        scratch_types=[
            pltpu.VMEM((chunk,), jnp.float32),
            pltpu.VMEM((chunk,), jnp.int32),
            pltpu.VMEM((16 * _STRIDE,), jnp.float32),
            pltpu.VMEM((_G,), jnp.float32),
            pltpu.VMEM((_G if vector_prev else 16,), jnp.float32),
            pltpu.VMEM_SHARED((16 * _G,), jnp.float32),
            pltpu.SemaphoreType.DMA,
            pltpu.SemaphoreType.DMA,
        ],
        compiler_params=pltpu.CompilerParams(needs_layout_passes=False),
    )


_HALF = 50176
_CHUNK_B = 3120
_LAST_B = _N - _HALF - (_NS - 1) * _CHUNK_B


def kernel(h, p, h_batch, W, b):
    ids = h_batch.astype(jnp.int32)
    s = _node_scores(h, p, W, 0, _NPAD // _BLK).reshape(_NPAD)
    return _make_seg(0, 0, _CHUNK, _LAST, False)(s, ids, b)


# speedup vs baseline: 1.0897x; 1.0224x over previous
import functools

import jax
import jax.numpy as jnp
from jax import lax
from jax.experimental import pallas as pl
from jax.experimental.pallas import tpu as pltpu
from jax.experimental.pallas import tpu_sc as plsc

_N = 100000
_H = 128
_G = 512
_BLK = 10240
_NPAD = 102400

_NS = 16
_CHUNK = 6400
_LAST = _N - (_NS - 1) * _CHUNK
_IT_FULL = _CHUNK // 16
_IT_LAST = _LAST // 16
_STRIDE = 513


def _scores_body(off_blocks, h_ref, p_ref, w_ref, o_ref):
    dn = (((1,), (1,)), ((), ()))
    s = (lax.dot_general(w_ref[:, :_H], h_ref[...], dn,
                         preferred_element_type=jnp.float32)
         + lax.dot_general(w_ref[:, _H:], p_ref[...], dn,
                           preferred_element_type=jnp.float32))
    i = pl.program_id(0) + off_blocks
    col = i * _BLK + lax.broadcasted_iota(jnp.int32, (1, _BLK), 1)
    o_ref[...] = jnp.where(col < _N, s, 0.0)


def _node_scores(h, p, W, off_blocks, nblocks):
    return pl.pallas_call(
        functools.partial(_scores_body, off_blocks),
        grid=(nblocks,),
        in_specs=[
            pl.BlockSpec((_BLK, _H), lambda i: (i + off_blocks, 0)),
            pl.BlockSpec((_BLK, _H), lambda i: (i + off_blocks, 0)),
            pl.BlockSpec((1, 2 * _H), lambda i: (0, 0)),
        ],
        out_specs=pl.BlockSpec((1, _BLK), lambda i: (0, i)),
        out_shape=jax.ShapeDtypeStruct((1, nblocks * _BLK), jnp.float32),
    )(h, p, W)


def _make_seg_body(sbase0, ibase0, chunk, last, vector_prev):
    it_full = chunk // 16
    it_last = last // 16

    def body(s_hbm, ids_hbm, prev_hbm, out_hbm, sv, iv, acc, accg, pv,
             shared, sem_s, sem_i):
        sid = lax.axis_index("s")
        sbase = sbase0 + sid * chunk
        ibase = ibase0 + sid * chunk
        is_last = sid == _NS - 1

        if last == chunk:
            pltpu.async_copy(s_hbm.at[pl.ds(sbase, chunk)], sv, sem_s)
            pltpu.async_copy(ids_hbm.at[pl.ds(ibase, chunk)], iv, sem_i)
        else:
            @pl.when(jnp.logical_not(is_last))
            def _():
                pltpu.async_copy(s_hbm.at[pl.ds(sbase, chunk)], sv, sem_s)
                pltpu.async_copy(ids_hbm.at[pl.ds(ibase, chunk)], iv, sem_i)

            @pl.when(is_last)
            def _():
                pltpu.async_copy(s_hbm.at[pl.ds(sbase, last)],
                                 sv.at[pl.ds(0, last)], sem_s)
                pltpu.async_copy(ids_hbm.at[pl.ds(ibase, last)],
                                 iv.at[pl.ds(0, last)], sem_i)

        zeros16 = jnp.zeros((16,), jnp.float32)

        def _zero(i, c):
            acc[pl.ds(i * 16, 16)] = zeros16
            return c

        lax.fori_loop(0, _STRIDE, _zero, 0, unroll=16)

        if last == chunk:
            pltpu.make_async_copy(s_hbm.at[pl.ds(sbase, chunk)], sv,
                                  sem_s).wait()
            pltpu.make_async_copy(ids_hbm.at[pl.ds(ibase, chunk)], iv,
                                  sem_i).wait()
        else:
            @pl.when(jnp.logical_not(is_last))
            def _():
                pltpu.make_async_copy(s_hbm.at[pl.ds(sbase, chunk)], sv,
                                      sem_s).wait()
                pltpu.make_async_copy(ids_hbm.at[pl.ds(ibase, chunk)], iv,
                                      sem_i).wait()

            @pl.when(is_last)
            def _():
                pltpu.make_async_copy(s_hbm.at[pl.ds(sbase, last)],
                                      sv.at[pl.ds(0, last)], sem_s).wait()
                pltpu.make_async_copy(ids_hbm.at[pl.ds(ibase, last)],
                                      iv.at[pl.ds(0, last)], sem_i).wait()

        lane_off = lax.broadcasted_iota(jnp.int32, (16,), 0) * _STRIDE

        def _scat(i, c):
            idx = iv[pl.ds(i * 16, 16)] + lane_off
            vals = sv[pl.ds(i * 16, 16)]
            plsc.addupdate_scatter(acc, [idx], vals)
            return c

        if last == chunk:
            lax.fori_loop(0, it_full, _scat, 0, unroll=8)
        else:
            @pl.when(jnp.logical_not(is_last))
            def _():
                lax.fori_loop(0, it_full, _scat, 0, unroll=8)

            @pl.when(is_last)
            def _():
                lax.fori_loop(0, it_last, _scat, 0, unroll=8)

        def _red(j, c):
            v = acc[pl.ds(j * 16, 16)]
            for l in range(1, 16):
                v = v + acc[pl.ds(l * _STRIDE + j * 16, 16)]
            accg[pl.ds(j * 16, 16)] = v
            return c

        lax.fori_loop(0, _G // 16, _red, 0)

        pltpu.sync_copy(accg, shared.at[pl.ds(sid * _G, _G)])
        plsc.subcore_barrier()

        t32 = sid * 32
        copies = [pltpu.make_async_copy(
            shared.at[pl.ds(l * _G + t32, 32)],
            acc.at[pl.ds(l * 32, 32)], sem_s) for l in range(16)]
        for c in copies:
            c.start()
        if vector_prev:
            pltpu.sync_copy(prev_hbm.at[pl.ds(t32, 32)], pv.at[pl.ds(0, 32)])
        else:
            pltpu.sync_copy(prev_hbm, pv.at[pl.ds(0, 1)])
        for c in copies:
            c.wait()
        bias = None if vector_prev else pv[pl.ds(0, 16)][0]
        for j in range(2):
            if vector_prev:
                v = pv[pl.ds(j * 16, 16)]
            else:
                v = jnp.zeros((16,), jnp.float32) + bias
            for l in range(16):
                v = v + acc[pl.ds(l * 32 + j * 16, 16)]
            accg[pl.ds(j * 16, 16)] = v
        pltpu.sync_copy(accg.at[pl.ds(0, 32)], out_hbm.at[pl.ds(t32, 32)])

    return body


def _make_seg(sbase0, ibase0, chunk, last, vector_prev):
    mesh = plsc.VectorSubcoreMesh(
        core_axis_name="c", subcore_axis_name="s", num_cores=1)
    return pl.kernel(
        _make_seg_body(sbase0, ibase0, chunk, last, vector_prev),
        out_type=jax.ShapeDtypeStruct((_G,), jnp.float32),
        mesh=mesh,
        scratch_types=[
            pltpu.VMEM((chunk,), jnp.float32),
            pltpu.VMEM((chunk,), jnp.int32),
            pltpu.VMEM((16 * _STRIDE,), jnp.float32),
            pltpu.VMEM((_G,), jnp.float32),
            pltpu.VMEM((_G if vector_prev else 16,), jnp.float32),
            pltpu.VMEM_SHARED((16 * _G,), jnp.float32),
            pltpu.SemaphoreType.DMA,
            pltpu.SemaphoreType.DMA,
        ],
        compiler_params=pltpu.CompilerParams(needs_layout_passes=False),
    )


_HALF = 50176
_CHUNK_B = 3120
_LAST_B = _N - _HALF - (_NS - 1) * _CHUNK_B


def kernel(h, p, h_batch, W, b):
    ids = h_batch.astype(jnp.int32)
    s = _node_scores(h, p, W, 0, _NPAD // _BLK).reshape(_NPAD)
    return _make_seg(0, 0, _CHUNK, _LAST, False)(s, ids, b)
